# baseline (speedup 1.0000x reference)
import functools

import jax
import jax.numpy as jnp
from jax import lax
from jax.experimental import pallas as pl
from jax.experimental.pallas import tpu as pltpu
from jax.experimental.pallas import tpu_sc as plsc

N = 10000
E = 320000
D = 128
NB = 5
KIN = 34
KPAD = 40
BN_EPS = 1e-5

BLK = 1000
NBLK = N // BLK

NC, NS = 2, 16
NW = NC * NS
EW = E // NW
CHUNK = 128
EWP = 10240
NCHUNK = EWP // CHUNK
PH = 2
CPP = NCHUNK // PH
TRASH = 128
NROWS = N + TRASH
NZW = 10
RPS = N // NZW


def _embed_body(xc_ref, wc_ref, b_ref, e_ref, h_ref, t_ref):
    h = jnp.dot(xc_ref[...], wc_ref[...],
                preferred_element_type=jnp.float32) + b_ref[...]
    h_ref[...] = h
    for t in range(NB):
        t_ref[t] = jnp.maximum(h + e_ref[t], 0.0)


def _embed(xc, wc, b, e0):
    return pl.pallas_call(
        _embed_body,
        grid=(NBLK,),
        in_specs=[
            pl.BlockSpec((BLK, KPAD), lambda i: (i, 0)),
            pl.BlockSpec((KPAD, D), lambda i: (0, 0)),
            pl.BlockSpec((1, D), lambda i: (0, 0)),
            pl.BlockSpec((NB, D), lambda i: (0, 0)),
        ],
        out_specs=[
            pl.BlockSpec((BLK, D), lambda i: (i, 0)),
            pl.BlockSpec((NB, BLK, D), lambda i: (0, i, 0)),
        ],
        out_shape=[
            jax.ShapeDtypeStruct((N, D), jnp.float32),
            jax.ShapeDtypeStruct((NB, N, D), jnp.float32),
        ],
    )(xc, wc, b, e0)


def _idx_body(src_ref, attr_ref, out_ref):
    out_ref[...] = attr_ref[...] * N + src_ref[...]


def _fuse_idx(src2d, attr2d):
    return pl.pallas_call(
        _idx_body,
        out_shape=jax.ShapeDtypeStruct(src2d.shape, jnp.int32),
    )(src2d, attr2d)


def _sc_body(t_hbm, aug_hbm, dst_hbm, zero_hbm, out_hbm,
             idx_v, dst_v, rows_v, acc_sh, sem0, sem1, sems0, sems1):
    cc = lax.axis_index("c")
    s = lax.axis_index("s")
    w = cc * NS + s
    @pl.when(s < NZW)
    def _():
        pltpu.sync_copy(zero_hbm, acc_sh.at[pl.ds(s * RPS, RPS)])
    plsc.subcore_barrier()

    buf0, buf1 = rows_v.at[0], rows_v.at[1]

    def body(c2, carry):
        ci = 2 * c2
        pltpu.make_async_copy(t_hbm.at[idx_v.at[ci]], buf0, sem0).wait()
        pltpu.async_copy(buf0, acc_sh.at[dst_v.at[ci]], sems0, add=True)
        pltpu.make_async_copy(t_hbm.at[idx_v.at[ci + 1]], buf1, sem1).wait()
        pltpu.async_copy(buf1, acc_sh.at[dst_v.at[ci + 1]], sems1, add=True)
        pltpu.make_async_copy(buf0, acc_sh.at[dst_v.at[ci]], sems0).wait()

        @pl.when(ci + 2 < CPP)
        def _():
            pltpu.async_copy(t_hbm.at[idx_v.at[ci + 2]], buf0, sem0)

        pltpu.make_async_copy(buf1, acc_sh.at[dst_v.at[ci + 1]], sems1).wait()

        @pl.when(ci + 3 < CPP)
        def _():
            pltpu.async_copy(t_hbm.at[idx_v.at[ci + 3]], buf1, sem1)

        return carry

    for p in range(PH):
        pltpu.sync_copy(aug_hbm.at[w, p], idx_v)
        pltpu.sync_copy(dst_hbm.at[w, p], dst_v)
        pltpu.async_copy(t_hbm.at[idx_v.at[0]], buf0, sem0)
        pltpu.async_copy(t_hbm.at[idx_v.at[1]], buf1, sem1)
        lax.fori_loop(0, CPP // 2, body, 0)
    plsc.subcore_barrier()

    @pl.when(s < NZW)
    def _():
        pltpu.sync_copy(acc_sh.at[pl.ds(s * RPS, RPS)],
                        out_hbm.at[cc, pl.ds(s * RPS, RPS)])


_sc_gather_scatter = functools.partial(
    pl.kernel,
    out_type=jax.ShapeDtypeStruct((NC, N, D), jnp.float32),
    mesh=plsc.VectorSubcoreMesh(core_axis_name="c", subcore_axis_name="s"),
    scratch_types=[
        pltpu.VMEM((CPP, CHUNK), jnp.int32),
        pltpu.VMEM((CPP, CHUNK), jnp.int32),
        pltpu.VMEM((2, CHUNK, D), jnp.float32),
        pltpu.VMEM_SHARED((NROWS, D), jnp.float32),
        pltpu.SemaphoreType.DMA,
        pltpu.SemaphoreType.DMA,
        pltpu.SemaphoreType.DMA,
        pltpu.SemaphoreType.DMA,
    ],
)(_sc_body)


def _mlp_body(h_ref, parts_ref, w1_ref, b1_ref, w2_ref, b2_ref,
              out_ref, stats_ref, acc):
    i = pl.program_id(0)
    z = h_ref[...] + parts_ref[0] + parts_ref[1]
    a = jnp.maximum(jnp.dot(z, w1_ref[...],
                            preferred_element_type=jnp.float32) + b1_ref[...],
                    0.0)
    o = jnp.dot(a, w2_ref[...], preferred_element_type=jnp.float32) + b2_ref[...]
    out_ref[...] = o

    @pl.when(i == 0)
    def _():
        acc[...] = jnp.zeros_like(acc)

    acc[0:1] += jnp.sum(o, axis=0, keepdims=True)
    acc[1:2] += jnp.sum(o * o, axis=0, keepdims=True)

    @pl.when(i == NBLK - 1)
    def _():
        stats_ref[...] = acc[...]


def _mlp(h, parts, w1, b1, w2, b2):
    return pl.pallas_call(
        _mlp_body,
        grid=(NBLK,),
        in_specs=[
            pl.BlockSpec((BLK, D), lambda i: (i, 0)),
            pl.BlockSpec((NC, BLK, D), lambda i: (0, i, 0)),
            pl.BlockSpec((D, 2 * D), lambda i: (0, 0)),
            pl.BlockSpec((1, 2 * D), lambda i: (0, 0)),
            pl.BlockSpec((2 * D, D), lambda i: (0, 0)),
            pl.BlockSpec((1, D), lambda i: (0, 0)),
        ],
        out_specs=[
            pl.BlockSpec((BLK, D), lambda i: (i, 0)),
            pl.BlockSpec((2, D), lambda i: (0, 0)),
        ],
        out_shape=[
            jax.ShapeDtypeStruct((N, D), jnp.float32),
            jax.ShapeDtypeStruct((2, D), jnp.float32),
        ],
        scratch_shapes=[pltpu.VMEM((2, D), jnp.float32)],
    )(h, parts, w1, b1, w2, b2)


def _bn_scale_shift(stats_ref, gamma_ref, beta_ref):
    mean = stats_ref[0:1] * (1.0 / N)
    var = stats_ref[1:2] * (1.0 / N) - mean * mean
    scale = gamma_ref[...] * lax.rsqrt(var + BN_EPS)
    shift = beta_ref[...] - mean * scale
    return scale, shift


def _normt_body(o_ref, stats_ref, gamma_ref, beta_ref, e_ref, h_ref, t_ref):
    scale, shift = _bn_scale_shift(stats_ref, gamma_ref, beta_ref)
    h = jnp.maximum(o_ref[...] * scale + shift, 0.0)
    h_ref[...] = h
    for t in range(NB):
        t_ref[t] = jnp.maximum(h + e_ref[t], 0.0)


def _normt(o, stats, gamma, beta, e_next):
    return pl.pallas_call(
        _normt_body,
        grid=(NBLK,),
        in_specs=[
            pl.BlockSpec((BLK, D), lambda i: (i, 0)),
            pl.BlockSpec((2, D), lambda i: (0, 0)),
            pl.BlockSpec((1, D), lambda i: (0, 0)),
            pl.BlockSpec((1, D), lambda i: (0, 0)),
            pl.BlockSpec((NB, D), lambda i: (0, 0)),
        ],
        out_specs=[
            pl.BlockSpec((BLK, D), lambda i: (i, 0)),
            pl.BlockSpec((NB, BLK, D), lambda i: (0, i, 0)),
        ],
        out_shape=[
            jax.ShapeDtypeStruct((N, D), jnp.float32),
            jax.ShapeDtypeStruct((NB, N, D), jnp.float32),
        ],
    )(o, stats, gamma, beta, e_next)


def _normf_body(o_ref, stats_ref, gamma_ref, beta_ref, h_ref):
    scale, shift = _bn_scale_shift(stats_ref, gamma_ref, beta_ref)
    h_ref[...] = o_ref[...] * scale + shift


def _normf(o, stats, gamma, beta):
    return pl.pallas_call(
        _normf_body,
        grid=(NBLK,),
        in_specs=[
            pl.BlockSpec((BLK, D), lambda i: (i, 0)),
            pl.BlockSpec((2, D), lambda i: (0, 0)),
            pl.BlockSpec((1, D), lambda i: (0, 0)),
            pl.BlockSpec((1, D), lambda i: (0, 0)),
        ],
        out_specs=pl.BlockSpec((BLK, D), lambda i: (i, 0)),
        out_shape=jax.ShapeDtypeStruct((N, D), jnp.float32),
    )(o, stats, gamma, beta)


def kernel(x, edge_index, edge_attr, instrument, fp,
           w_atom, b_atom, w_inst, b_inst, edge_emb,
           mlp_w1, mlp_b1, mlp_w2, mlp_b2, bn_gamma, bn_beta):
    src = edge_index[0]
    dst = edge_index[1]

    xc = jnp.concatenate(
        [x, instrument, jnp.zeros((N, KPAD - KIN), jnp.float32)], axis=1)
    wc = jnp.concatenate(
        [w_atom, w_inst, jnp.zeros((KPAD - KIN, D), jnp.float32)], axis=0)
    b = (b_atom + b_inst).reshape(1, D)

    h, t_tab = _embed(xc, wc, b, edge_emb[0])

    aug = _fuse_idx(src.reshape(E // D, D), edge_attr.reshape(E // D, D))
    padi = jnp.arange(EWP - EW, dtype=jnp.int32)
    aug = jnp.concatenate(
        [aug.reshape(NW, EW),
         jnp.broadcast_to(padi[None], (NW, EWP - EW))],
        axis=1).reshape(NW, PH, CPP, CHUNK)
    dst3 = jnp.concatenate(
        [dst.reshape(NW, EW),
         jnp.broadcast_to(N + (padi % TRASH)[None], (NW, EWP - EW))],
        axis=1).reshape(NW, PH, CPP, CHUNK)
    zeros = jnp.zeros((RPS, D), jnp.float32)

    for layer in range(3):
        parts = _sc_gather_scatter(t_tab.reshape(NB * N, D), aug, dst3, zeros)
        o, stats = _mlp(h, parts,
                        mlp_w1[layer], mlp_b1[layer].reshape(1, 2 * D),
                        mlp_w2[layer], mlp_b2[layer].reshape(1, D))
        gamma = bn_gamma[layer].reshape(1, D)
        beta = bn_beta[layer].reshape(1, D)
        if layer < 2:
            h, t_tab = _normt(o, stats, gamma, beta, edge_emb[layer + 1])
        else:
            h = _normf(o, stats, gamma, beta)
    return h


# speedup vs baseline: 1.0865x; 1.0865x over previous
import functools

import jax
import jax.numpy as jnp
from jax import lax
from jax.experimental import pallas as pl
from jax.experimental.pallas import tpu as pltpu
from jax.experimental.pallas import tpu_sc as plsc

N = 10000
E = 320000
D = 128
NB = 5
KIN = 34
KPAD = 40
BN_EPS = 1e-5

BLK = 1000
NBLK = N // BLK

NC, NS = 2, 16
NW = NC * NS
EW = E // NW
CHUNK = 128
EWP = 10240
NCHUNK = EWP // CHUNK
PH = 2
CPP = NCHUNK // PH
TRASH = 128
NROWS = N + TRASH
NZW = 10
RPS = N // NZW


def _bf16_dot(a, b):
    return jnp.dot(a.astype(jnp.bfloat16), b.astype(jnp.bfloat16),
                   preferred_element_type=jnp.float32)


def _embed_body(xc_ref, wc_ref, b_ref, e_ref, h_ref, t_ref):
    h = _bf16_dot(xc_ref[...], wc_ref[...]) + b_ref[...]
    h_ref[...] = h
    for t in range(NB):
        t_ref[t] = jnp.maximum(h + e_ref[t], 0.0)


def _embed(xc, wc, b, e0):
    return pl.pallas_call(
        _embed_body,
        grid=(NBLK,),
        in_specs=[
            pl.BlockSpec((BLK, KPAD), lambda i: (i, 0)),
            pl.BlockSpec((KPAD, D), lambda i: (0, 0)),
            pl.BlockSpec((1, D), lambda i: (0, 0)),
            pl.BlockSpec((NB, D), lambda i: (0, 0)),
        ],
        out_specs=[
            pl.BlockSpec((BLK, D), lambda i: (i, 0)),
            pl.BlockSpec((NB, BLK, D), lambda i: (0, i, 0)),
        ],
        out_shape=[
            jax.ShapeDtypeStruct((N, D), jnp.float32),
            jax.ShapeDtypeStruct((NB, N, D), jnp.float32),
        ],
    )(xc, wc, b, e0)


def _idx_body(src_ref, attr_ref, out_ref):
    out_ref[...] = attr_ref[...] * N + src_ref[...]


def _fuse_idx(src2d, attr2d):
    return pl.pallas_call(
        _idx_body,
        out_shape=jax.ShapeDtypeStruct(src2d.shape, jnp.int32),
    )(src2d, attr2d)


def _sc_body(t_hbm, aug_hbm, dst_hbm, zero_hbm, out_hbm,
             idx_v, dst_v, rows_v, acc_sh, sem0, sem1):
    cc = lax.axis_index("c")
    s = lax.axis_index("s")
    w = cc * NS + s
    @pl.when(s < NZW)
    def _():
        pltpu.sync_copy(zero_hbm, acc_sh.at[pl.ds(s * RPS, RPS)])
    plsc.subcore_barrier()

    buf0, buf1 = rows_v.at[0], rows_v.at[1]

    def body(c2, carry):
        ci = 2 * c2
        pltpu.make_async_copy(t_hbm.at[idx_v.at[ci]], buf0, sem0).wait()
        pltpu.async_copy(t_hbm.at[idx_v.at[ci + 1]], buf1, sem1)
        pltpu.sync_copy(buf0, acc_sh.at[dst_v.at[ci]], add=True)
        pltpu.make_async_copy(t_hbm.at[idx_v.at[ci + 1]], buf1, sem1).wait()

        @pl.when(ci + 2 < CPP)
        def _():
            pltpu.async_copy(t_hbm.at[idx_v.at[ci + 2]], buf0, sem0)

        pltpu.sync_copy(buf1, acc_sh.at[dst_v.at[ci + 1]], add=True)
        return carry

    for p in range(PH):
        pltpu.sync_copy(aug_hbm.at[w, p], idx_v)
        pltpu.sync_copy(dst_hbm.at[w, p], dst_v)
        pltpu.async_copy(t_hbm.at[idx_v.at[0]], buf0, sem0)
        lax.fori_loop(0, CPP // 2, body, 0)
    plsc.subcore_barrier()

    @pl.when(s < NZW)
    def _():
        pltpu.sync_copy(acc_sh.at[pl.ds(s * RPS, RPS)],
                        out_hbm.at[cc, pl.ds(s * RPS, RPS)])


_sc_gather_scatter = functools.partial(
    pl.kernel,
    out_type=jax.ShapeDtypeStruct((NC, N, D), jnp.float32),
    mesh=plsc.VectorSubcoreMesh(core_axis_name="c", subcore_axis_name="s"),
    scratch_types=[
        pltpu.VMEM((CPP, CHUNK), jnp.int32),
        pltpu.VMEM((CPP, CHUNK), jnp.int32),
        pltpu.VMEM((2, CHUNK, D), jnp.float32),
        pltpu.VMEM_SHARED((NROWS, D), jnp.float32),
        pltpu.SemaphoreType.DMA,
        pltpu.SemaphoreType.DMA,
    ],
)(_sc_body)


def _mlp_body(h_ref, parts_ref, w1_ref, b1_ref, w2_ref, b2_ref,
              out_ref, stats_ref, acc):
    i = pl.program_id(0)
    z = h_ref[...] + parts_ref[0] + parts_ref[1]
    a = jnp.maximum(_bf16_dot(z, w1_ref[...]) + b1_ref[...], 0.0)
    o = _bf16_dot(a, w2_ref[...]) + b2_ref[...]
    out_ref[...] = o

    @pl.when(i == 0)
    def _():
        acc[...] = jnp.zeros_like(acc)

    acc[0:1] += jnp.sum(o, axis=0, keepdims=True)
    acc[1:2] += jnp.sum(o * o, axis=0, keepdims=True)

    @pl.when(i == NBLK - 1)
    def _():
        stats_ref[...] = acc[...]


def _mlp(h, parts, w1, b1, w2, b2):
    return pl.pallas_call(
        _mlp_body,
        grid=(NBLK,),
        in_specs=[
            pl.BlockSpec((BLK, D), lambda i: (i, 0)),
            pl.BlockSpec((NC, BLK, D), lambda i: (0, i, 0)),
            pl.BlockSpec((D, 2 * D), lambda i: (0, 0)),
            pl.BlockSpec((1, 2 * D), lambda i: (0, 0)),
            pl.BlockSpec((2 * D, D), lambda i: (0, 0)),
            pl.BlockSpec((1, D), lambda i: (0, 0)),
        ],
        out_specs=[
            pl.BlockSpec((BLK, D), lambda i: (i, 0)),
            pl.BlockSpec((2, D), lambda i: (0, 0)),
        ],
        out_shape=[
            jax.ShapeDtypeStruct((N, D), jnp.float32),
            jax.ShapeDtypeStruct((2, D), jnp.float32),
        ],
        scratch_shapes=[pltpu.VMEM((2, D), jnp.float32)],
    )(h, parts, w1, b1, w2, b2)


def _bn_scale_shift(stats_ref, gamma_ref, beta_ref):
    mean = stats_ref[0:1] * (1.0 / N)
    var = stats_ref[1:2] * (1.0 / N) - mean * mean
    scale = gamma_ref[...] * lax.rsqrt(var + BN_EPS)
    shift = beta_ref[...] - mean * scale
    return scale, shift


def _normt_body(o_ref, stats_ref, gamma_ref, beta_ref, e_ref, h_ref, t_ref):
    scale, shift = _bn_scale_shift(stats_ref, gamma_ref, beta_ref)
    h = jnp.maximum(o_ref[...] * scale + shift, 0.0)
    h_ref[...] = h
    for t in range(NB):
        t_ref[t] = jnp.maximum(h + e_ref[t], 0.0)


def _normt(o, stats, gamma, beta, e_next):
    return pl.pallas_call(
        _normt_body,
        grid=(NBLK,),
        in_specs=[
            pl.BlockSpec((BLK, D), lambda i: (i, 0)),
            pl.BlockSpec((2, D), lambda i: (0, 0)),
            pl.BlockSpec((1, D), lambda i: (0, 0)),
            pl.BlockSpec((1, D), lambda i: (0, 0)),
            pl.BlockSpec((NB, D), lambda i: (0, 0)),
        ],
        out_specs=[
            pl.BlockSpec((BLK, D), lambda i: (i, 0)),
            pl.BlockSpec((NB, BLK, D), lambda i: (0, i, 0)),
        ],
        out_shape=[
            jax.ShapeDtypeStruct((N, D), jnp.float32),
            jax.ShapeDtypeStruct((NB, N, D), jnp.float32),
        ],
    )(o, stats, gamma, beta, e_next)


def _normf_body(o_ref, stats_ref, gamma_ref, beta_ref, h_ref):
    scale, shift = _bn_scale_shift(stats_ref, gamma_ref, beta_ref)
    h_ref[...] = o_ref[...] * scale + shift


def _normf(o, stats, gamma, beta):
    return pl.pallas_call(
        _normf_body,
        grid=(NBLK,),
        in_specs=[
            pl.BlockSpec((BLK, D), lambda i: (i, 0)),
            pl.BlockSpec((2, D), lambda i: (0, 0)),
            pl.BlockSpec((1, D), lambda i: (0, 0)),
            pl.BlockSpec((1, D), lambda i: (0, 0)),
        ],
        out_specs=pl.BlockSpec((BLK, D), lambda i: (i, 0)),
        out_shape=jax.ShapeDtypeStruct((N, D), jnp.float32),
    )(o, stats, gamma, beta)


def kernel(x, edge_index, edge_attr, instrument, fp,
           w_atom, b_atom, w_inst, b_inst, edge_emb,
           mlp_w1, mlp_b1, mlp_w2, mlp_b2, bn_gamma, bn_beta):
    src = edge_index[0]
    dst = edge_index[1]

    xc = jnp.concatenate(
        [x, instrument, jnp.zeros((N, KPAD - KIN), jnp.float32)], axis=1)
    wc = jnp.concatenate(
        [w_atom, w_inst, jnp.zeros((KPAD - KIN, D), jnp.float32)], axis=0)
    b = (b_atom + b_inst).reshape(1, D)

    h, t_tab = _embed(xc, wc, b, edge_emb[0])

    aug = _fuse_idx(src.reshape(E // D, D), edge_attr.reshape(E // D, D))
    padi = jnp.arange(EWP - EW, dtype=jnp.int32)
    aug = jnp.concatenate(
        [aug.reshape(NW, EW),
         jnp.broadcast_to(padi[None], (NW, EWP - EW))],
        axis=1).reshape(NW, PH, CPP, CHUNK)
    dst3 = jnp.concatenate(
        [dst.reshape(NW, EW),
         jnp.broadcast_to(N + (padi % TRASH)[None], (NW, EWP - EW))],
        axis=1).reshape(NW, PH, CPP, CHUNK)
    zeros = jnp.zeros((RPS, D), jnp.float32)

    for layer in range(3):
        parts = _sc_gather_scatter(t_tab.reshape(NB * N, D), aug, dst3, zeros)
        o, stats = _mlp(h, parts,
                        mlp_w1[layer], mlp_b1[layer].reshape(1, 2 * D),
                        mlp_w2[layer], mlp_b2[layer].reshape(1, D))
        gamma = bn_gamma[layer].reshape(1, D)
        beta = bn_beta[layer].reshape(1, D)
        if layer < 2:
            h, t_tab = _normt(o, stats, gamma, beta, edge_emb[layer + 1])
        else:
            h = _normf(o, stats, gamma, beta)
    return h
